# baseline (speedup 1.0000x reference)
import functools

import jax
import jax.numpy as jnp
from jax import lax
from jax.experimental import pallas as pl
from jax.experimental.pallas import tpu as pltpu
from jax.experimental.pallas import tpu_sc as plsc

N = 100000
D = 128
S = 512
NC = 2
NS = 16
NW = NC * NS
BLK = 128
NB = N // BLK
TAIL = N - NB * BLK
SEG_PER_TILE = S // NS
BASE_BLOCKS = NB // NW
EXTRA_WORKERS = NB - BASE_BLOCKS * NW
MAXB = BASE_BLOCKS + 1


def _sc_partials(x, batch):
    mesh = plsc.VectorSubcoreMesh(core_axis_name="c", subcore_axis_name="s")

    @functools.partial(
        pl.kernel,
        out_type=[
            jax.ShapeDtypeStruct((NC, 2, S, D), jnp.float32),
            jax.ShapeDtypeStruct((NW, S), jnp.float32),
        ],
        mesh=mesh,
        compiler_params=pltpu.CompilerParams(needs_layout_passes=False,
                                             use_tc_tiling_on_sc=False),
        scratch_types=[
            pltpu.VMEM((2, BLK, D), jnp.float32),
            pltpu.VMEM((MAXB, BLK), jnp.int32),
            pltpu.VMEM((TAIL, D), jnp.float32),
            pltpu.VMEM((TAIL,), jnp.int32),
            pltpu.VMEM((S,), jnp.float32),
            pltpu.VMEM((SEG_PER_TILE, D), jnp.float32),
            pltpu.VMEM_SHARED((S, D), jnp.float32),
            pltpu.VMEM_SHARED((S, D), jnp.float32),
            pltpu.SemaphoreType.DMA((2,)),
            pltpu.SemaphoreType.DMA((2,)),
            pltpu.SemaphoreType.DMA,
        ],
    )
    def sc_kernel(x_hbm, b_hbm, sum_out, cnt_out,
                  xbufs, ids_all, xt, ids_t, hist, zbuf, acc_a, acc_b,
                  ld_sems, sc_sems, id_sem):
        c = lax.axis_index("c")
        s = lax.axis_index("s")
        wid = s * NC + c

        sb = BASE_BLOCKS * wid + jnp.minimum(wid, EXTRA_WORKERS)
        nblk = BASE_BLOCKS + jnp.where(wid < EXTRA_WORKERS, 1, 0)

        for k in range(MAXB):
            @pl.when(k < nblk)
            def _stage_ids():
                pltpu.async_copy(b_hbm.at[pl.ds((sb + k) * BLK, BLK)],
                                 ids_all.at[k], id_sem)

        for p in range(2):
            pltpu.async_copy(x_hbm.at[pl.ds((sb + p) * BLK, BLK)],
                             xbufs.at[p], ld_sems.at[p])

        zeros16 = jnp.zeros((16,), jnp.float32)

        def zrow(i, carry):
            def zcol(j, carry2):
                zbuf[i, pl.ds(j * 16, 16)] = zeros16
                return carry2
            return lax.fori_loop(0, D // 16, zcol, carry)
        lax.fori_loop(0, SEG_PER_TILE, zrow, 0)

        def zh(i, carry):
            hist[pl.ds(i * 16, 16)] = zeros16
            return carry
        lax.fori_loop(0, S // 16, zh, 0)

        pltpu.sync_copy(zbuf, acc_a.at[pl.ds(s * SEG_PER_TILE, SEG_PER_TILE)])
        pltpu.sync_copy(zbuf, acc_b.at[pl.ds(s * SEG_PER_TILE, SEG_PER_TILE)])
        plsc.subcore_barrier()

        for k in range(MAXB):
            @pl.when(k < nblk)
            def _drain_ids():
                pltpu.make_async_copy(b_hbm.at[pl.ds((sb + k) * BLK, BLK)],
                                      ids_all.at[k], id_sem).wait()

        ones = jnp.full((16,), 1.0, jnp.float32)

        def pair(i, carry):
            for p in range(2):
                k = 2 * i + p
                dst = acc_a if p == 0 else acc_b

                @pl.when(k < nblk)
                def _start():
                    pltpu.make_async_copy(
                        x_hbm.at[pl.ds((sb + k) * BLK, BLK)],
                        xbufs.at[p], ld_sems.at[p]).wait()
                    pltpu.async_copy(xbufs.at[p], dst.at[ids_all.at[k]],
                                     sc_sems.at[p], add=True)

                    def grp(g, carry2):
                        idv = ids_all[k, pl.ds(g * 16, 16)]
                        plsc.addupdate_scatter(hist, [idv], ones)
                        return carry2
                    lax.fori_loop(0, BLK // 16, grp, 0)

            for p in range(2):
                k = 2 * i + p
                dst = acc_a if p == 0 else acc_b

                @pl.when(k < nblk)
                def _drain():
                    pltpu.make_async_copy(xbufs.at[p], dst.at[ids_all.at[k]],
                                          sc_sems.at[p]).wait()

                    @pl.when(k + 2 < nblk)
                    def _next_load():
                        pltpu.async_copy(
                            x_hbm.at[pl.ds((sb + k + 2) * BLK, BLK)],
                            xbufs.at[p], ld_sems.at[p])
            return carry
        lax.fori_loop(0, (MAXB + 1) // 2, pair, 0)

        @pl.when(wid == NW - 1)
        def _tail():
            base = NB * BLK
            pltpu.sync_copy(b_hbm.at[pl.ds(base, TAIL)], ids_t)
            pltpu.sync_copy(x_hbm.at[pl.ds(base, TAIL)], xt)
            pltpu.sync_copy(xt, acc_a.at[ids_t], add=True)

            def grp(g, carry):
                idv = ids_t[pl.ds(g * 16, 16)]
                plsc.addupdate_scatter(hist, [idv], ones)
                return carry
            lax.fori_loop(0, TAIL // 16, grp, 0)

        pltpu.sync_copy(hist, cnt_out.at[wid])
        plsc.subcore_barrier()
        pltpu.sync_copy(acc_a.at[pl.ds(s * SEG_PER_TILE, SEG_PER_TILE)],
                        sum_out.at[c, 0, pl.ds(s * SEG_PER_TILE, SEG_PER_TILE)])
        pltpu.sync_copy(acc_b.at[pl.ds(s * SEG_PER_TILE, SEG_PER_TILE)],
                        sum_out.at[c, 1, pl.ds(s * SEG_PER_TILE, SEG_PER_TILE)])

    return sc_kernel(x, batch)


def _combine(partial_sums, partial_counts):
    def body(sp_ref, cn_ref, o_ref):
        total = ((sp_ref[0, 0] + sp_ref[0, 1])
                 + (sp_ref[1, 0] + sp_ref[1, 1]))
        cnt = jnp.maximum(jnp.sum(cn_ref[...], axis=0), 1.0)
        o_ref[...] = total / cnt[:, None]

    return pl.pallas_call(
        body,
        out_shape=jax.ShapeDtypeStruct((S, D), jnp.float32),
    )(partial_sums, partial_counts)


def kernel(x, batch):
    batch = batch.astype(jnp.int32)
    partial_sums, partial_counts = _sc_partials(x, batch)
    return _combine(partial_sums, partial_counts)
